# baseline (speedup 1.0000x reference)
import functools
import jax
import jax.numpy as jnp
from jax import lax
from jax.experimental import pallas as pl
from jax.experimental.pallas import tpu as pltpu
from jax.experimental.pallas import tpu_sc as plsc

_B = 8
_CH = 255
_G = 76
_NA = 3
_ATTRS = 85
_STRIDE = 8.0
_ANCH_W = (10.0, 16.0, 33.0)
_ANCH_H = (13.0, 30.0, 23.0)
_NC = 2
_NS = 16
_NW = _NC * _NS
_L = 16
_QPB = _G // 2
_NCHUNKS = _B * _QPB
_ORPC = _NA * _G * 2


def _sc_body(x_hbm, out_hbm, inbuf, outbuf):
    wid = lax.axis_index("s") * _NC + lax.axis_index("c")
    iota = lax.broadcasted_iota(jnp.int32, (_L,), 0)
    onef = jnp.full((_L,), 1.0, jnp.float32)
    _JOFF = (0, 16, 32, 48, 60)
    idx_r = [[(iota + jo + _G * g) * _NA for jo in _JOFF] for g in range(2)]
    colvf = [(iota + jo).astype(jnp.float32) for jo in _JOFF]

    def sigmoid_vec(r, g, j):
        v = inbuf[r, g, pl.ds(_JOFF[j], _L)]
        e = jnp.exp(-v)
        return onef / (onef + e)

    def chunk_body(t, _):
        ch = wid + _NW * t

        @pl.when(ch < _NCHUNKS)
        def _():
            b = ch // _QPB
            q = ch - b * _QPB
            gr0 = q * 2
            pltpu.sync_copy(x_hbm.at[b, :, gr0, :], inbuf.at[:, 0, :])
            pltpu.sync_copy(x_hbm.at[b, :, gr0 + 1, :], inbuf.at[:, 1, :])

            for g in range(2):
                grf = jnp.full((_L,), 1.0, jnp.float32) * (
                    (gr0 + g).astype(jnp.float32))
                for a in range(_NA):
                    base = _ATTRS * a
                    av = iota * 0 + a

                    for j in range(5):
                        rv = idx_r[g][j] + av
                        vx = (sigmoid_vec(base + 0, g, j) + colvf[j]) * _STRIDE
                        plsc.store_scatter(outbuf, [rv, iota * 0], vx)
                        vy = (sigmoid_vec(base + 1, g, j) + grf) * _STRIDE
                        plsc.store_scatter(outbuf, [rv, iota * 0 + 1], vy)
                        vw = jnp.exp(inbuf[base + 2, g, pl.ds(_JOFF[j], _L)])
                        plsc.store_scatter(outbuf, [rv, iota * 0 + 2],
                                           vw * _ANCH_W[a])
                        vh = jnp.exp(inbuf[base + 3, g, pl.ds(_JOFF[j], _L)])
                        plsc.store_scatter(outbuf, [rv, iota * 0 + 3],
                                           vh * _ANCH_H[a])

                    def attr_body(k, _, g=g, a=a, base=base, av=av):
                        kv = iota * 0 + k
                        for j in range(5):
                            vv = sigmoid_vec(base + k, g, j)
                            plsc.store_scatter(outbuf, [idx_r[g][j] + av, kv],
                                               vv)
                        return 0

                    lax.fori_loop(4, _ATTRS, attr_body, 0)

            pltpu.sync_copy(outbuf, out_hbm.at[b, pl.ds(q * _ORPC, _ORPC), :])

        return 0

    lax.fori_loop(0, (_NCHUNKS + _NW - 1) // _NW, chunk_body, 0)


def kernel(x):
    mesh = plsc.VectorSubcoreMesh(core_axis_name="c", subcore_axis_name="s")
    f = functools.partial(
        pl.kernel,
        out_type=jax.ShapeDtypeStruct((_B, _NA * _G * _G, _ATTRS), jnp.float32),
        mesh=mesh,
        compiler_params=pltpu.CompilerParams(needs_layout_passes=False),
        scratch_types=[
            pltpu.VMEM((_CH, 2, _G), jnp.float32),
            pltpu.VMEM((_ORPC, _ATTRS), jnp.float32),
        ],
    )(_sc_body)
    return f(x)


# speedup vs baseline: 3.6305x; 3.6305x over previous
import jax
import jax.numpy as jnp
from jax.experimental import pallas as pl
from jax.experimental.pallas import tpu as pltpu

_G = 76
_CELLS = _G * _G
_NA = 3
_ATTRS = 85
_STRIDE = 8.0
_ANCH_W = (10.0, 16.0, 33.0)
_ANCH_H = (13.0, 30.0, 23.0)


_CC = 1536
_NCK = 4


def _dl_kernel(x_ref, o_ref):
    ci = pl.program_id(1)
    off = pl.multiple_of(ci * _CC, 128)
    v = x_ref[0, :, pl.ds(off, _CC)]
    r = jax.lax.broadcasted_iota(jnp.int32, v.shape, 0)
    j = jax.lax.broadcasted_iota(jnp.int32, v.shape, 1) + off
    colf = (j % _G).astype(jnp.float32)
    rowf = (j // _G).astype(jnp.float32)
    a = r // _ATTRS
    attr = r - a * _ATTRS
    aw = jnp.where(a == 0, _ANCH_W[0], jnp.where(a == 1, _ANCH_W[1], _ANCH_W[2]))
    ah = jnp.where(a == 0, _ANCH_H[0], jnp.where(a == 1, _ANCH_H[1], _ANCH_H[2]))
    is_w = attr == 2
    is_wh = is_w | (attr == 3)
    e = jnp.exp(jnp.where(is_wh, v, -v))
    val = jnp.where(is_wh, e * jnp.where(is_w, aw, ah), 1.0 / (1.0 + e))
    off = jnp.where(attr == 0, colf, jnp.where(attr == 1, rowf, 0.0))
    scale = jnp.where(attr < 2, _STRIDE, 1.0)
    val = (val + off) * scale
    o_ref[0] = val.T


def kernel(x):
    b = x.shape[0]
    xf = x.reshape(b, _NA * _ATTRS, _CELLS)
    out = pl.pallas_call(
        _dl_kernel,
        grid=(b, _NCK),
        in_specs=[pl.BlockSpec((1, _NA * _ATTRS, _NCK * _CC),
                               lambda bi, ci: (bi, 0, 0))],
        out_specs=pl.BlockSpec((1, _CC, _NA * _ATTRS),
                               lambda bi, ci: (bi, ci, 0)),
        out_shape=jax.ShapeDtypeStruct((b, _CELLS, _NA * _ATTRS), jnp.float32),
        compiler_params=pltpu.CompilerParams(
            dimension_semantics=("parallel", "arbitrary"),
        ),
    )(xf)
    return out.reshape(b, _CELLS * _NA, _ATTRS)
